# baseline (device time: 53828 ns/iter reference)
import jax
import jax.numpy as jnp
from jax import lax
from jax.experimental import pallas as pl
from jax.experimental.pallas import tpu as pltpu

N_DEV = 4
BLK = 1024
N_BLOCKS = 8


def _carry_kernel(x):
    m, n = x.shape

    def body(x_ref, out_ref, acc_ref, total_ref, recv_ref, send_sems, recv_sems):
        i = pl.program_id(0)
        my = lax.axis_index("i")

        @pl.when(i == 0)
        def _():
            acc_ref[...] = jnp.ones((8, n), jnp.float32)

        h = x_ref[...]
        for s in (512, 256, 128, 64, 32, 16, 8):
            h = h[:s, :] * h[s:2 * s, :]
        acc_ref[...] = acc_ref[...] * h

        @pl.when(i == N_BLOCKS - 1)
        def _exchange():
            t8 = acc_ref[...]
            for s in (4, 2, 1):
                t8 = t8[:s, :] * t8[s:2 * s, :]
            total_ref[...] = t8


            for j in range(N_DEV - 1):
                @pl.when(my <= j)
                def _(j=j):
                    recv_ref[j, :, :] = jnp.ones((1, n), jnp.float32)

            for j in range(N_DEV - 1):
                @pl.when(my == j)
                def _(j=j):
                    rdmas = [
                        pltpu.make_async_remote_copy(
                            src_ref=total_ref,
                            dst_ref=recv_ref.at[j],
                            send_sem=send_sems.at[t],
                            recv_sem=recv_sems.at[j],
                            device_id=(t,),
                            device_id_type=pl.DeviceIdType.MESH,
                        )
                        for t in range(j + 1, N_DEV)
                    ]
                    for r in rdmas:
                        r.start()
                    for r in rdmas:
                        r.wait_send()

            for j in range(N_DEV - 1):
                @pl.when(my > j)
                def _(j=j):
                    recv = pltpu.make_async_remote_copy(
                        src_ref=total_ref,
                        dst_ref=recv_ref.at[j],
                        send_sem=send_sems.at[0],
                        recv_sem=recv_sems.at[j],
                        device_id=(0,),
                        device_id_type=pl.DeviceIdType.MESH,
                    )
                    recv.wait_recv()

            out_ref[...] = recv_ref[0] * recv_ref[1] * recv_ref[2]

    return pl.pallas_call(
        body,
        grid=(N_BLOCKS,),
        out_shape=jax.ShapeDtypeStruct((1, n), jnp.float32),
        in_specs=[pl.BlockSpec((BLK, n), lambda i: (i, 0))],
        out_specs=pl.BlockSpec((1, n), lambda i: (0, 0)),
        scratch_shapes=[
            pltpu.VMEM((8, n), jnp.float32),
            pltpu.VMEM((1, n), jnp.float32),
            pltpu.VMEM((N_DEV - 1, 1, n), jnp.float32),
            pltpu.SemaphoreType.DMA((N_DEV,)),
            pltpu.SemaphoreType.DMA((N_DEV - 1,)),
        ],
        compiler_params=pltpu.CompilerParams(
            dimension_semantics=("arbitrary",),
            vmem_limit_bytes=60 * 1024 * 1024,
        ),
    )(x)


def _scan_kernel(x, carry0):
    m, n = x.shape

    def body(x_ref, c0_ref, out_ref, carry_ref):
        i = pl.program_id(0)

        @pl.when(i == 0)
        def _():
            carry_ref[...] = c0_ref[...]

        one = lambda *shape: jnp.ones(shape, jnp.float32)
        r = x_ref[...].reshape(128, 8, n)
        for s in (1, 2, 4):
            r = r * jnp.concatenate([one(128, s, n), r[:, :8 - s, :]], axis=1)
        t = r[:, 7:8, :].reshape(16, 8, n)
        for s in (1, 2, 4):
            t = t * jnp.concatenate([one(16, s, n), t[:, :8 - s, :]], axis=1)
        u = t[:, 7:8, :]
        for s in (1, 2, 4, 8):
            u = u * jnp.concatenate([one(s, 1, n), u[:16 - s, :, :]], axis=0)
        exc_u = jnp.concatenate([one(1, 1, n), u[:15]], axis=0)
        exc_t = jnp.concatenate([one(16, 1, n), t[:, :7, :]], axis=1)
        scale = (exc_t * exc_u).reshape(128, 1, n)
        y = (r * (scale * carry_ref[...].reshape(1, 1, n))).reshape(BLK, n)
        out_ref[...] = y
        carry_ref[...] = y[BLK - 1:BLK, :]

    return pl.pallas_call(
        body,
        grid=(m // BLK,),
        out_shape=jax.ShapeDtypeStruct((m, n), jnp.float32),
        in_specs=[
            pl.BlockSpec((BLK, n), lambda i: (i, 0)),
            pl.BlockSpec(memory_space=pltpu.MemorySpace.VMEM),
        ],
        out_specs=pl.BlockSpec((BLK, n), lambda i: (i, 0)),
        scratch_shapes=[pltpu.VMEM((1, n), jnp.float32)],
        compiler_params=pltpu.CompilerParams(
            dimension_semantics=("arbitrary",),
            vmem_limit_bytes=60 * 1024 * 1024,
        ),
    )(x, carry0)


def kernel(x):
    carry0 = _carry_kernel(x)
    return _scan_kernel(x, carry0)


# device time: 51980 ns/iter; 1.0356x vs baseline; 1.0356x over previous
import jax
import jax.numpy as jnp
from jax import lax
from jax.experimental import pallas as pl
from jax.experimental.pallas import tpu as pltpu

N_DEV = 4
BLK = 1024
N_BLOCKS = 8


def _carry_kernel(x):
    m, n = x.shape

    def body(x_ref, out_ref, acc_ref, total_ref, recv_ref, send_sems, recv_sems):
        i = pl.program_id(0)
        my = lax.axis_index("i")

        @pl.when(i == 0)
        def _():
            acc_ref[...] = jnp.ones((8, n), jnp.float32)
            barrier_sem = pltpu.get_barrier_semaphore()
            for d in range(N_DEV):
                @pl.when(my != d)
                def _(d=d):
                    pl.semaphore_signal(
                        barrier_sem, inc=1,
                        device_id=(d,), device_id_type=pl.DeviceIdType.MESH,
                    )
            pl.semaphore_wait(barrier_sem, N_DEV - 1)

        h = x_ref[...]
        for s in (512, 256, 128, 64, 32, 16, 8):
            h = h[:s, :] * h[s:2 * s, :]
        acc_ref[...] = acc_ref[...] * h

        @pl.when(i == N_BLOCKS - 1)
        def _exchange():
            t8 = acc_ref[...]
            for s in (4, 2, 1):
                t8 = t8[:s, :] * t8[s:2 * s, :]
            total_ref[...] = t8

            for j in range(N_DEV - 1):
                @pl.when(my <= j)
                def _(j=j):
                    recv_ref[j, :, :] = jnp.ones((1, n), jnp.float32)

            for j in range(N_DEV - 1):
                @pl.when(my == j)
                def _(j=j):
                    rdmas = [
                        pltpu.make_async_remote_copy(
                            src_ref=total_ref,
                            dst_ref=recv_ref.at[j],
                            send_sem=send_sems.at[t],
                            recv_sem=recv_sems.at[j],
                            device_id=(t,),
                            device_id_type=pl.DeviceIdType.MESH,
                        )
                        for t in range(j + 1, N_DEV)
                    ]
                    for r in rdmas:
                        r.start()
                    for r in rdmas:
                        r.wait_send()

            for j in range(N_DEV - 1):
                @pl.when(my > j)
                def _(j=j):
                    recv = pltpu.make_async_remote_copy(
                        src_ref=total_ref,
                        dst_ref=recv_ref.at[j],
                        send_sem=send_sems.at[0],
                        recv_sem=recv_sems.at[j],
                        device_id=(0,),
                        device_id_type=pl.DeviceIdType.MESH,
                    )
                    recv.wait_recv()

            out_ref[...] = recv_ref[0] * recv_ref[1] * recv_ref[2]

    return pl.pallas_call(
        body,
        grid=(N_BLOCKS,),
        out_shape=jax.ShapeDtypeStruct((1, n), jnp.float32),
        in_specs=[pl.BlockSpec((BLK, n), lambda i: (i, 0))],
        out_specs=pl.BlockSpec((1, n), lambda i: (0, 0)),
        scratch_shapes=[
            pltpu.VMEM((8, n), jnp.float32),
            pltpu.VMEM((1, n), jnp.float32),
            pltpu.VMEM((N_DEV - 1, 1, n), jnp.float32),
            pltpu.SemaphoreType.DMA((N_DEV,)),
            pltpu.SemaphoreType.DMA((N_DEV - 1,)),
        ],
        compiler_params=pltpu.CompilerParams(
            collective_id=0,
            dimension_semantics=("arbitrary",),
            vmem_limit_bytes=60 * 1024 * 1024,
        ),
    )(x)


def _scan_kernel(x, carry0):
    m, n = x.shape

    def body(x_ref, c0_ref, out_ref, carry_ref):
        i = pl.program_id(0)

        @pl.when(i == 0)
        def _():
            carry_ref[...] = c0_ref[...]

        one = lambda *shape: jnp.ones(shape, jnp.float32)
        r = x_ref[...].reshape(128, 8, n)
        for s in (1, 2, 4):
            r = r * jnp.concatenate([one(128, s, n), r[:, :8 - s, :]], axis=1)
        t = r[:, 7:8, :].reshape(16, 8, n)
        for s in (1, 2, 4):
            t = t * jnp.concatenate([one(16, s, n), t[:, :8 - s, :]], axis=1)
        u = t[:, 7:8, :]
        for s in (1, 2, 4, 8):
            u = u * jnp.concatenate([one(s, 1, n), u[:16 - s, :, :]], axis=0)
        exc_u = jnp.concatenate([one(1, 1, n), u[:15]], axis=0)
        exc_t = jnp.concatenate([one(16, 1, n), t[:, :7, :]], axis=1)
        scale = (exc_t * exc_u).reshape(128, 1, n)
        y = (r * (scale * carry_ref[...].reshape(1, 1, n))).reshape(BLK, n)
        out_ref[...] = y
        carry_ref[...] = y[BLK - 1:BLK, :]

    return pl.pallas_call(
        body,
        grid=(m // BLK,),
        out_shape=jax.ShapeDtypeStruct((m, n), jnp.float32),
        in_specs=[
            pl.BlockSpec((BLK, n), lambda i: (i, 0)),
            pl.BlockSpec(memory_space=pltpu.MemorySpace.VMEM),
        ],
        out_specs=pl.BlockSpec((BLK, n), lambda i: (i, 0)),
        scratch_shapes=[pltpu.VMEM((1, n), jnp.float32)],
        compiler_params=pltpu.CompilerParams(
            dimension_semantics=("arbitrary",),
            vmem_limit_bytes=60 * 1024 * 1024,
        ),
    )(x, carry0)


def kernel(x):
    carry0 = _carry_kernel(x)
    return _scan_kernel(x, carry0)
